# baseline (speedup 1.0000x reference)
import jax
import jax.numpy as jnp
from jax.experimental import pallas as pl
from jax.experimental.pallas import tpu as pltpu

_HW_BLK = 2048


def _prelu_body(route_ref, w_ref, f_ref, x_ref, o_ref):
    slope = (w_ref[0, 0] + f_ref[0])[:, None]
    xv = x_ref[0]
    o_ref[0] = jnp.where(xv >= 0, xv, slope * xv)


def kernel(input, route_index, weight, weight_fact):
    B, C, H, W = input.shape
    HW = H * W
    x = input.reshape(B, C, HW)
    routes = route_index.astype(jnp.int32)
    w3 = weight.reshape(weight.shape[0], 1, C)

    grid = (B, HW // _HW_BLK)
    grid_spec = pltpu.PrefetchScalarGridSpec(
        num_scalar_prefetch=1,
        grid=grid,
        in_specs=[
            pl.BlockSpec((1, 1, C), lambda b, j, r: (r[b], 0, 0)),
            pl.BlockSpec((1, C), lambda b, j, r: (0, 0)),
            pl.BlockSpec((1, C, _HW_BLK), lambda b, j, r: (b, 0, j)),
        ],
        out_specs=pl.BlockSpec((1, C, _HW_BLK), lambda b, j, r: (b, 0, j)),
    )
    out = pl.pallas_call(
        _prelu_body,
        grid_spec=grid_spec,
        out_shape=jax.ShapeDtypeStruct((B, C, HW), jnp.float32),
        compiler_params=pltpu.CompilerParams(
            dimension_semantics=("arbitrary", "arbitrary"),
        ),
    )(routes, w3, weight_fact, x)
    return out.reshape(B, C, H, W)
